# baseline (speedup 1.0000x reference)
import functools

import jax
import jax.numpy as jnp
from jax import lax
from jax.experimental import pallas as pl
from jax.experimental.pallas import tpu as pltpu
from jax.experimental.pallas import tpu_sc as plsc

_L = 16


def _rsqrt16(d):
    i = plsc.bitcast(d, jnp.int32)
    yi = jnp.int32(0x5F3759DF) - lax.shift_right_logical(i, 1)
    y = plsc.bitcast(yi, jnp.float32)
    for _ in range(3):
        y = y * (1.5 - 0.5 * d * y * y)
    return y


def _sc_propagate(xf, srcs, dsts, ews, N, HQ, NG, G, NPAD, K):
    NS = 16
    NAT = NPAD // NS
    NSLOT = 5
    mesh = plsc.VectorSubcoreMesh(core_axis_name="c", subcore_axis_name="s")

    @functools.partial(
        pl.kernel,
        out_type=jax.ShapeDtypeStruct((4 * NPAD, HQ), jnp.float32),
        mesh=mesh,
        compiler_params=pltpu.CompilerParams(needs_layout_passes=False,
                                             use_tc_tiling_on_sc=False),
        scratch_types=dict(
            sbuf=pltpu.HBM((4 * NPAD, HQ), jnp.float32),
            pbuf=pltpu.HBM((2, NS, NPAD), jnp.float32),
            dst_v=pltpu.VMEM((NG, G), jnp.int32),
            ew_v=pltpu.VMEM((NG, G), jnp.float32),
            gi_v=pltpu.VMEM((NG, G), jnp.int32),
            deg_v=pltpu.VMEM((NPAD,), jnp.float32),
            stg_v=pltpu.VMEM((NAT,), jnp.float32),
            dacc_v=pltpu.VMEM((NAT,), jnp.float32),
            srg_v=pltpu.VMEM((NSLOT, G), jnp.int32),
            rowbuf=pltpu.VMEM((NSLOT, G, HQ), jnp.float32),
            acc=pltpu.VMEM_SHARED((NPAD, HQ), jnp.float32),
            dsh=pltpu.VMEM_SHARED((NPAD,), jnp.float32),
            gsem0=pltpu.SemaphoreType.DMA,
            gsem1=pltpu.SemaphoreType.DMA,
            gsem2=pltpu.SemaphoreType.DMA,
            gsem3=pltpu.SemaphoreType.DMA,
            gsem4=pltpu.SemaphoreType.DMA,
            ssem0=pltpu.SemaphoreType.DMA,
            ssem1=pltpu.SemaphoreType.DMA,
            ssem2=pltpu.SemaphoreType.DMA,
            ssem3=pltpu.SemaphoreType.DMA,
            ssem4=pltpu.SemaphoreType.DMA,
        ),
    )
    def prop(xf_h, srcs_h, dsts_h, ews_h, out_h, *, sbuf, pbuf,
             dst_v, ew_v, gi_v, deg_v, stg_v, dacc_v, srg_v, rowbuf,
             acc, dsh, gsem0, gsem1, gsem2, gsem3, gsem4,
             ssem0, ssem1, ssem2, ssem3, ssem4):
        c = lax.axis_index("c")
        s = lax.axis_index("s")
        z16 = jnp.zeros((_L,), jnp.float32)

        pltpu.sync_copy(dsts_h.at[s], dst_v)
        pltpu.sync_copy(ews_h.at[s], ew_v)

        @pl.loop(0, NPAD // _L)
        def _(r):
            deg_v[pl.ds(r * _L, _L)] = z16

        @pl.loop(0, NAT // _L)
        def _(r):
            dacc_v[pl.ds(r * _L, _L)] = z16

        @pl.loop(0, NG)
        def _(g):
            for k in range(G // _L):
                sl = pl.ds(k * _L, _L)
                t16 = dst_v[g, sl]
                w16 = ew_v[g, sl]
                plsc.addupdate_scatter(deg_v, [t16], w16)

        pltpu.sync_copy(deg_v, pbuf.at[c, s])
        plsc.subcore_barrier()
        for t in range(NS):
            pltpu.sync_copy(pbuf.at[c, t, pl.ds(s * NAT, NAT)], stg_v)

            @pl.loop(0, NAT // _L)
            def _(r):
                sl = pl.ds(r * _L, _L)
                dacc_v[sl] = dacc_v[sl] + stg_v[sl]

        @pl.loop(0, NAT // _L)
        def _(r):
            sl = pl.ds(r * _L, _L)
            dacc_v[sl] = _rsqrt16(dacc_v[sl])

        pltpu.sync_copy(dacc_v, dsh.at[pl.ds(s * NAT, NAT)])
        plsc.subcore_barrier()
        pltpu.sync_copy(dsh, deg_v)

        gsems = (gsem0, gsem1, gsem2, gsem3, gsem4)
        ssems = (ssem0, ssem1, ssem2, ssem3, ssem4)

        for q in range(2):
            pltpu.async_copy(srcs_h.at[s, q], srg_v.at[q], gsems[q])

        @pl.loop(0, NG, step=NSLOT)
        def _(g):
            for par in range(NSLOT):
                gc = g + par
                pltpu.make_async_copy(srcs_h.at[s, 0], srg_v.at[par],
                                      gsems[par]).wait()
                nxt = (par + 2) % NSLOT

                @pl.when(gc + 2 < NG)
                def _():
                    pltpu.async_copy(srcs_h.at[s, gc + 2],
                                     srg_v.at[nxt], gsems[nxt])

                for k in range(G // _L):
                    sl = pl.ds(k * _L, _L)
                    s16 = srg_v[par, sl]
                    t16 = dst_v[gc, sl]
                    w16 = ew_v[gc, sl]
                    di_s = plsc.load_gather(deg_v, [s16])
                    di_t = plsc.load_gather(deg_v, [t16])
                    ew_v[gc, sl] = di_s * w16 * di_t
                    gi_v[gc, sl] = s16 * 4 + c * 2
        plsc.subcore_barrier()

        def upd_gi(f):
            @pl.loop(0, NG)
            def _(g):
                for k in range(G // _L):
                    sl = pl.ds(k * _L, _L)
                    gi_v[g, sl] = f(gi_v[g, sl])

        def run_pass(src_ref, out_ref, slab):
            @pl.loop(0, G)
            def _(r):
                for j in range(HQ // _L):
                    rowbuf[0, r, pl.ds(j * _L, _L)] = z16

            for z in range(NAT // G):
                pltpu.sync_copy(rowbuf.at[0],
                                acc.at[pl.ds(s * NAT + z * G, G)])
            plsc.subcore_barrier()

            for q in range(2):
                pltpu.async_copy(src_ref.at[gi_v.at[q]], rowbuf.at[q],
                                 gsems[q])

            @pl.loop(0, NG, step=NSLOT)
            def _(g):
                for par in range(NSLOT):
                    gc = g + par
                    pltpu.make_async_copy(
                        xf_h.at[pl.ds(0, G)], rowbuf.at[par],
                        gsems[par]).wait()

                    nxt = (par + 2) % NSLOT

                    @pl.when(gc >= 3)
                    def _():
                        pltpu.make_async_copy(
                            xf_h.at[pl.ds(0, G)], rowbuf.at[nxt],
                            ssems[nxt]).wait()

                    @pl.when(gc + 2 < NG)
                    def _():
                        pltpu.async_copy(
                            src_ref.at[gi_v.at[gc + 2]],
                            rowbuf.at[nxt], gsems[nxt])

                    gsplat = jnp.full((_L,), gc, jnp.int32)

                    @pl.loop(0, G, step=8)
                    def _(i):
                        for ii in range(8):
                            nb = plsc.load_gather(
                                ew_v,
                                [gsplat, jnp.full((_L,), i + ii, jnp.int32)])
                            for j in range(HQ // _L):
                                sl = pl.ds(j * _L, _L)
                                rowbuf[par, i + ii, sl] = \
                                    rowbuf[par, i + ii, sl] * nb

                    pltpu.async_copy(rowbuf.at[par], acc.at[dst_v.at[gc]],
                                     ssems[par], add=True)
            for gg in (NG - 3, NG - 2, NG - 1):
                pltpu.make_async_copy(xf_h.at[pl.ds(0, G)],
                                      rowbuf.at[gg % NSLOT],
                                      ssems[gg % NSLOT]).wait()
            plsc.subcore_barrier()
            pltpu.sync_copy(
                acc.at[pl.ds(s * NAT, NAT)],
                out_ref.at[pl.ds(slab * NPAD + s * NAT, NAT)])
            plsc.subcore_barrier()

        run_pass(xf_h, sbuf, c * 2)
        upd_gi(lambda v: v + 1)
        run_pass(xf_h, sbuf, c * 2 + 1)
        upd_gi(lambda v: lax.shift_right_logical(v - (c * 2 + 1), 2)
               + c * 2 * NPAD)
        run_pass(sbuf, sbuf, c * 2)
        upd_gi(lambda v: v + NPAD)
        run_pass(sbuf, sbuf, c * 2 + 1)
        upd_gi(lambda v: v - NPAD)
        run_pass(sbuf, out_h, c * 2)
        upd_gi(lambda v: v + NPAD)
        run_pass(sbuf, out_h, c * 2 + 1)

    return prop(xf, srcs, dsts, ews)


def _mm_body(h_ref, w_ref, b_ref, o_ref, *, HQ):
    dn = (((1,), (1,)), ((), ()))
    o = b_ref[...]
    for q in range(4):
        o = o + lax.dot_general(h_ref[q], w_ref[:, q * HQ:(q + 1) * HQ], dn,
                                preferred_element_type=jnp.float32)
    o_ref[...] = o


def kernel(x, edge_index, edge_attr, W, b):
    N, D = x.shape
    E = edge_index.shape[1]
    HQ = D // 4
    NS, G = 16, 128

    src = edge_index[0].astype(jnp.int32)
    dst = edge_index[1].astype(jnp.int32)
    loop = jnp.arange(N, dtype=jnp.int32)

    E2 = E + N
    NG = -(-E2 // (NS * G))
    NG += (-NG) % 5
    E2p = NS * NG * G
    pad = E2p - E2
    zi = jnp.zeros((pad,), jnp.int32)
    zf = jnp.zeros((pad,), x.dtype)
    src2 = jnp.concatenate([src, loop, zi]).reshape(NS, NG, G)
    dst2 = jnp.concatenate([dst, loop, zi]).reshape(NS, NG, G)
    ew2 = jnp.concatenate([edge_attr, jnp.ones((N,), x.dtype), zf]).reshape(NS, NG, G)

    NPAD = -(-N // 2048) * 2048

    xf = x.reshape(4 * N, HQ)
    h3 = _sc_propagate(xf, src2, dst2, ew2, N, HQ, NG, G, NPAD, K=3)

    BN = 1000
    out = pl.pallas_call(
        functools.partial(_mm_body, HQ=HQ),
        grid=(N // BN,),
        in_specs=[
            pl.BlockSpec((4, BN, HQ), lambda i: (0, i, 0)),
            pl.BlockSpec((D, D), lambda i: (0, 0)),
            pl.BlockSpec((1, D), lambda i: (0, 0)),
        ],
        out_specs=pl.BlockSpec((BN, D), lambda i: (i, 0)),
        out_shape=jax.ShapeDtypeStruct((N, D), jnp.float32),
    )(h3.reshape(4, NPAD, HQ), W, b.reshape(1, D))
    return out


# speedup vs baseline: 1.2679x; 1.2679x over previous
import functools

import jax
import jax.numpy as jnp
from jax import lax
from jax.experimental import pallas as pl
from jax.experimental.pallas import tpu as pltpu
from jax.experimental.pallas import tpu_sc as plsc

_L = 16


def _rsqrt16(d):
    i = plsc.bitcast(d, jnp.int32)
    yi = jnp.int32(0x5F3759DF) - lax.shift_right_logical(i, 1)
    y = plsc.bitcast(yi, jnp.float32)
    for _ in range(3):
        y = y * (1.5 - 0.5 * d * y * y)
    return y


def _sc_propagate(xf, srcs, dsts, ews, N, HQ, NG, G, NPAD, K):
    NS = 16
    NAT = NPAD // NS
    mesh = plsc.VectorSubcoreMesh(core_axis_name="c", subcore_axis_name="s")

    @functools.partial(
        pl.kernel,
        out_type=jax.ShapeDtypeStruct((4 * NPAD, HQ), jnp.float32),
        mesh=mesh,
        compiler_params=pltpu.CompilerParams(needs_layout_passes=False,
                                             use_tc_tiling_on_sc=False),
        scratch_types=dict(
            sbuf=pltpu.HBM((4 * NPAD, HQ), jnp.float32),
            pbuf=pltpu.HBM((2, NS, NPAD), jnp.float32),
            src_v=pltpu.VMEM((NG, G), jnp.int32),
            dst_v=pltpu.VMEM((NG, G), jnp.int32),
            ew_v=pltpu.VMEM((NG, G), jnp.float32),
            gi_v=pltpu.VMEM((NG, G), jnp.int32),
            deg_v=pltpu.VMEM((NPAD,), jnp.float32),
            stg_v=pltpu.VMEM((NAT,), jnp.float32),
            dacc_v=pltpu.VMEM((NAT,), jnp.float32),
            rowbuf=pltpu.VMEM((4, G, HQ), jnp.float32),
            acc=pltpu.VMEM_SHARED((NPAD, HQ), jnp.float32),
            dsh=pltpu.VMEM_SHARED((NPAD,), jnp.float32),
            gsem0=pltpu.SemaphoreType.DMA,
            gsem1=pltpu.SemaphoreType.DMA,
            gsem2=pltpu.SemaphoreType.DMA,
            gsem3=pltpu.SemaphoreType.DMA,
            ssem0=pltpu.SemaphoreType.DMA,
            ssem1=pltpu.SemaphoreType.DMA,
            ssem2=pltpu.SemaphoreType.DMA,
            ssem3=pltpu.SemaphoreType.DMA,
        ),
    )
    def prop(xf_h, srcs_h, dsts_h, ews_h, out_h, *, sbuf, pbuf, src_v,
             dst_v, ew_v, gi_v, deg_v, stg_v, dacc_v, rowbuf,
             acc, dsh, gsem0, gsem1, gsem2, gsem3,
             ssem0, ssem1, ssem2, ssem3):
        c = lax.axis_index("c")
        s = lax.axis_index("s")
        z16 = jnp.zeros((_L,), jnp.float32)

        pltpu.sync_copy(srcs_h.at[s], src_v)
        pltpu.sync_copy(dsts_h.at[s], dst_v)
        pltpu.sync_copy(ews_h.at[s], ew_v)

        @pl.loop(0, NPAD // _L)
        def _(r):
            deg_v[pl.ds(r * _L, _L)] = z16

        @pl.loop(0, NAT // _L)
        def _(r):
            dacc_v[pl.ds(r * _L, _L)] = z16

        @pl.loop(0, NG)
        def _(g):
            for k in range(G // _L):
                sl = pl.ds(k * _L, _L)
                t16 = dst_v[g, sl]
                w16 = ew_v[g, sl]
                plsc.addupdate_scatter(deg_v, [t16], w16)

        pltpu.sync_copy(deg_v, pbuf.at[c, s])
        plsc.subcore_barrier()
        for t in range(NS):
            pltpu.sync_copy(pbuf.at[c, t, pl.ds(s * NAT, NAT)], stg_v)

            @pl.loop(0, NAT // _L)
            def _(r):
                sl = pl.ds(r * _L, _L)
                dacc_v[sl] = dacc_v[sl] + stg_v[sl]

        @pl.loop(0, NAT // _L)
        def _(r):
            sl = pl.ds(r * _L, _L)
            dacc_v[sl] = _rsqrt16(dacc_v[sl])

        pltpu.sync_copy(dacc_v, dsh.at[pl.ds(s * NAT, NAT)])
        plsc.subcore_barrier()
        pltpu.sync_copy(dsh, deg_v)

        @pl.loop(0, NG)
        def _(g):
            for k in range(G // _L):
                sl = pl.ds(k * _L, _L)
                s16 = src_v[g, sl]
                t16 = dst_v[g, sl]
                w16 = ew_v[g, sl]
                di_s = plsc.load_gather(deg_v, [s16])
                di_t = plsc.load_gather(deg_v, [t16])
                ew_v[g, sl] = di_s * w16 * di_t
        plsc.subcore_barrier()

        gsems = (gsem0, gsem1, gsem2, gsem3)
        ssems = (ssem0, ssem1, ssem2, ssem3)
        NSLOT = 4

        def run_pass(src_ref, out_ref, idx_mul, slab_stride, slab):
            @pl.loop(0, NG)
            def _(g):
                for k in range(G // _L):
                    sl = pl.ds(k * _L, _L)
                    gi_v[g, sl] = src_v[g, sl] * idx_mul + slab * slab_stride

            @pl.loop(0, G)
            def _(r):
                for j in range(HQ // _L):
                    rowbuf[0, r, pl.ds(j * _L, _L)] = z16

            for z in range(NAT // G):
                pltpu.sync_copy(rowbuf.at[0],
                                acc.at[pl.ds(s * NAT + z * G, G)])
            plsc.subcore_barrier()

            for q in range(2):
                pltpu.async_copy(src_ref.at[gi_v.at[q]], rowbuf.at[q],
                                 gsems[q])

            @pl.loop(0, NG, step=NSLOT)
            def _(g):
                for par in range(NSLOT):
                    gc = g + par
                    pltpu.make_async_copy(
                        xf_h.at[pl.ds(0, G)], rowbuf.at[par],
                        gsems[par]).wait()

                    nxt = (par + 2) % NSLOT

                    @pl.when(gc >= 2)
                    def _():
                        pltpu.make_async_copy(
                            xf_h.at[pl.ds(0, G)], rowbuf.at[nxt],
                            ssems[nxt]).wait()

                    @pl.when(gc + 2 < NG)
                    def _():
                        pltpu.async_copy(
                            src_ref.at[gi_v.at[gc + 2]],
                            rowbuf.at[nxt], gsems[nxt])

                    gsplat = jnp.full((_L,), gc, jnp.int32)

                    @pl.loop(0, G, step=8)
                    def _(i):
                        for ii in range(8):
                            nb = plsc.load_gather(
                                ew_v,
                                [gsplat, jnp.full((_L,), i + ii, jnp.int32)])
                            for j in range(HQ // _L):
                                sl = pl.ds(j * _L, _L)
                                rowbuf[par, i + ii, sl] = \
                                    rowbuf[par, i + ii, sl] * nb

                    pltpu.async_copy(rowbuf.at[par], acc.at[dst_v.at[gc]],
                                     ssems[par], add=True)
            for gg in (NG - 2, NG - 1):
                pltpu.make_async_copy(xf_h.at[pl.ds(0, G)],
                                      rowbuf.at[gg % NSLOT],
                                      ssems[gg % NSLOT]).wait()
            plsc.subcore_barrier()
            pltpu.sync_copy(
                acc.at[pl.ds(s * NAT, NAT)],
                out_ref.at[pl.ds(slab * NPAD + s * NAT, NAT)])
            plsc.subcore_barrier()

        for hop in range(K):
            for p in range(2):
                out_ref = sbuf if hop < K - 1 else out_h
                if hop == 0:
                    run_pass(xf_h, out_ref, 4, 1, c * 2 + p)
                else:
                    run_pass(sbuf, out_ref, 1, NPAD, c * 2 + p)

    return prop(xf, srcs, dsts, ews)


def _mm_body(h_ref, w_ref, b_ref, o_ref, *, HQ):
    dn = (((1,), (1,)), ((), ()))
    o = b_ref[...]
    for q in range(4):
        o = o + lax.dot_general(h_ref[q], w_ref[:, q * HQ:(q + 1) * HQ], dn,
                                preferred_element_type=jnp.float32)
    o_ref[...] = o


def kernel(x, edge_index, edge_attr, W, b):
    N, D = x.shape
    E = edge_index.shape[1]
    HQ = D // 4
    NS, G = 16, 128

    src = edge_index[0].astype(jnp.int32)
    dst = edge_index[1].astype(jnp.int32)
    loop = jnp.arange(N, dtype=jnp.int32)

    E2 = E + N
    per_tile_groups = -(-E2 // (NS * G))
    NG = per_tile_groups + (per_tile_groups % 2)
    E2p = NS * NG * G
    pad = E2p - E2
    zi = jnp.zeros((pad,), jnp.int32)
    zf = jnp.zeros((pad,), x.dtype)
    src2 = jnp.concatenate([src, loop, zi]).reshape(NS, NG, G)
    dst2 = jnp.concatenate([dst, loop, zi]).reshape(NS, NG, G)
    ew2 = jnp.concatenate([edge_attr, jnp.ones((N,), x.dtype), zf]).reshape(NS, NG, G)

    NPAD = -(-N // 2048) * 2048

    xf = x.reshape(4 * N, HQ)
    h3 = _sc_propagate(xf, src2, dst2, ew2, N, HQ, NG, G, NPAD, K=3)

    BN = 1000
    out = pl.pallas_call(
        functools.partial(_mm_body, HQ=HQ),
        grid=(N // BN,),
        in_specs=[
            pl.BlockSpec((4, BN, HQ), lambda i: (0, i, 0)),
            pl.BlockSpec((D, D), lambda i: (0, 0)),
            pl.BlockSpec((1, D), lambda i: (0, 0)),
        ],
        out_specs=pl.BlockSpec((BN, D), lambda i: (i, 0)),
        out_shape=jax.ShapeDtypeStruct((N, D), jnp.float32),
    )(h3.reshape(4, NPAD, HQ), W, b.reshape(1, D))
    return out


# speedup vs baseline: 1.2702x; 1.0018x over previous
import functools

import jax
import jax.numpy as jnp
from jax import lax
from jax.experimental import pallas as pl
from jax.experimental.pallas import tpu as pltpu
from jax.experimental.pallas import tpu_sc as plsc

_L = 16


def _rsqrt16(d):
    i = plsc.bitcast(d, jnp.int32)
    yi = jnp.int32(0x5F3759DF) - lax.shift_right_logical(i, 1)
    y = plsc.bitcast(yi, jnp.float32)
    for _ in range(3):
        y = y * (1.5 - 0.5 * d * y * y)
    return y


def _sc_propagate(xf, srcs, dsts, ews, N, HQ, NG, G, NPAD, K):
    NS = 16
    NAT = NPAD // NS
    mesh = plsc.VectorSubcoreMesh(core_axis_name="c", subcore_axis_name="s")

    @functools.partial(
        pl.kernel,
        out_type=jax.ShapeDtypeStruct((4 * NPAD, HQ), jnp.float32),
        mesh=mesh,
        compiler_params=pltpu.CompilerParams(needs_layout_passes=False,
                                             use_tc_tiling_on_sc=False),
        scratch_types=dict(
            sbuf=pltpu.HBM((4 * NPAD, HQ), jnp.float32),
            pbuf=pltpu.HBM((2, NS, NPAD), jnp.float32),
            src_v=pltpu.VMEM((NG, G), jnp.int32),
            dst_v=pltpu.VMEM((NG, G), jnp.int32),
            ew_v=pltpu.VMEM((NG, G), jnp.float32),
            gi_v=pltpu.VMEM((NG, G), jnp.int32),
            deg_v=pltpu.VMEM((NPAD,), jnp.float32),
            stg_v=pltpu.VMEM((NAT,), jnp.float32),
            dacc_v=pltpu.VMEM((NAT,), jnp.float32),
            rowbuf=pltpu.VMEM((4, G, HQ), jnp.float32),
            acc=pltpu.VMEM_SHARED((NPAD, HQ), jnp.float32),
            dsh=pltpu.VMEM_SHARED((NPAD,), jnp.float32),
            gsem0=pltpu.SemaphoreType.DMA,
            gsem1=pltpu.SemaphoreType.DMA,
            gsem2=pltpu.SemaphoreType.DMA,
            gsem3=pltpu.SemaphoreType.DMA,
            ssem0=pltpu.SemaphoreType.DMA,
            ssem1=pltpu.SemaphoreType.DMA,
            ssem2=pltpu.SemaphoreType.DMA,
            ssem3=pltpu.SemaphoreType.DMA,
        ),
    )
    def prop(xf_h, srcs_h, dsts_h, ews_h, out_h, *, sbuf, pbuf, src_v,
             dst_v, ew_v, gi_v, deg_v, stg_v, dacc_v, rowbuf,
             acc, dsh, gsem0, gsem1, gsem2, gsem3,
             ssem0, ssem1, ssem2, ssem3):
        c = lax.axis_index("c")
        s = lax.axis_index("s")
        z16 = jnp.zeros((_L,), jnp.float32)

        pltpu.sync_copy(srcs_h.at[s], src_v)
        pltpu.sync_copy(dsts_h.at[s], dst_v)
        pltpu.sync_copy(ews_h.at[s], ew_v)

        @pl.loop(0, NPAD // _L)
        def _(r):
            deg_v[pl.ds(r * _L, _L)] = z16

        @pl.loop(0, NAT // _L)
        def _(r):
            dacc_v[pl.ds(r * _L, _L)] = z16

        @pl.loop(0, NG)
        def _(g):
            for k in range(G // _L):
                sl = pl.ds(k * _L, _L)
                t16 = dst_v[g, sl]
                w16 = ew_v[g, sl]
                plsc.addupdate_scatter(deg_v, [t16], w16)

        pltpu.sync_copy(deg_v, pbuf.at[c, s])
        plsc.subcore_barrier()
        for t in range(NS):
            pltpu.sync_copy(pbuf.at[c, t, pl.ds(s * NAT, NAT)], stg_v)

            @pl.loop(0, NAT // _L)
            def _(r):
                sl = pl.ds(r * _L, _L)
                dacc_v[sl] = dacc_v[sl] + stg_v[sl]

        @pl.loop(0, NAT // _L)
        def _(r):
            sl = pl.ds(r * _L, _L)
            dacc_v[sl] = _rsqrt16(dacc_v[sl])

        pltpu.sync_copy(dacc_v, dsh.at[pl.ds(s * NAT, NAT)])
        plsc.subcore_barrier()
        pltpu.sync_copy(dsh, deg_v)

        @pl.loop(0, NG)
        def _(g):
            for k in range(G // _L):
                sl = pl.ds(k * _L, _L)
                s16 = src_v[g, sl]
                t16 = dst_v[g, sl]
                w16 = ew_v[g, sl]
                di_s = plsc.load_gather(deg_v, [s16])
                di_t = plsc.load_gather(deg_v, [t16])
                ew_v[g, sl] = di_s * w16 * di_t
        plsc.subcore_barrier()

        gsems = (gsem0, gsem1, gsem2, gsem3)
        ssems = (ssem0, ssem1, ssem2, ssem3)
        NSLOT = 4

        def run_pass(src_ref, out_ref, idx_mul, slab_stride, slab):
            @pl.loop(0, NG)
            def _(g):
                for k in range(G // _L):
                    sl = pl.ds(k * _L, _L)
                    gi_v[g, sl] = src_v[g, sl] * idx_mul + slab * slab_stride

            @pl.loop(0, G)
            def _(r):
                for j in range(HQ // _L):
                    rowbuf[0, r, pl.ds(j * _L, _L)] = z16

            zsems = (ssems[0], ssems[1], ssems[2], ssems[3], gsems[3])
            for z in range(NAT // G):
                pltpu.async_copy(rowbuf.at[0],
                                 acc.at[pl.ds(s * NAT + z * G, G)],
                                 zsems[z])
            for z in range(NAT // G):
                pltpu.make_async_copy(xf_h.at[pl.ds(0, G)], rowbuf.at[0],
                                      zsems[z]).wait()
            plsc.subcore_barrier()

            for q in range(2):
                pltpu.async_copy(src_ref.at[gi_v.at[q]], rowbuf.at[q],
                                 gsems[q])

            @pl.loop(0, NG, step=NSLOT)
            def _(g):
                for par in range(NSLOT):
                    gc = g + par
                    pltpu.make_async_copy(
                        xf_h.at[pl.ds(0, G)], rowbuf.at[par],
                        gsems[par]).wait()

                    nxt = (par + 2) % NSLOT

                    @pl.when(gc >= 2)
                    def _():
                        pltpu.make_async_copy(
                            xf_h.at[pl.ds(0, G)], rowbuf.at[nxt],
                            ssems[nxt]).wait()

                    @pl.when(gc + 2 < NG)
                    def _():
                        pltpu.async_copy(
                            src_ref.at[gi_v.at[gc + 2]],
                            rowbuf.at[nxt], gsems[nxt])

                    gsplat = jnp.full((_L,), gc, jnp.int32)

                    @pl.loop(0, G, step=8)
                    def _(i):
                        for ii in range(8):
                            nb = plsc.load_gather(
                                ew_v,
                                [gsplat, jnp.full((_L,), i + ii, jnp.int32)])
                            for j in range(HQ // _L):
                                sl = pl.ds(j * _L, _L)
                                rowbuf[par, i + ii, sl] = \
                                    rowbuf[par, i + ii, sl] * nb

                    pltpu.async_copy(rowbuf.at[par], acc.at[dst_v.at[gc]],
                                     ssems[par], add=True)
            for gg in (NG - 2, NG - 1):
                pltpu.make_async_copy(xf_h.at[pl.ds(0, G)],
                                      rowbuf.at[gg % NSLOT],
                                      ssems[gg % NSLOT]).wait()
            plsc.subcore_barrier()
            pltpu.sync_copy(
                acc.at[pl.ds(s * NAT, NAT)],
                out_ref.at[pl.ds(slab * NPAD + s * NAT, NAT)])
            plsc.subcore_barrier()

        for hop in range(K):
            for p in range(2):
                out_ref = sbuf if hop < K - 1 else out_h
                if hop == 0:
                    run_pass(xf_h, out_ref, 4, 1, c * 2 + p)
                else:
                    run_pass(sbuf, out_ref, 1, NPAD, c * 2 + p)

    return prop(xf, srcs, dsts, ews)


def _mm_body(h_ref, w_ref, b_ref, o_ref, *, HQ):
    dn = (((1,), (1,)), ((), ()))
    o = b_ref[...]
    for q in range(4):
        o = o + lax.dot_general(h_ref[q], w_ref[:, q * HQ:(q + 1) * HQ], dn,
                                preferred_element_type=jnp.float32)
    o_ref[...] = o


def kernel(x, edge_index, edge_attr, W, b):
    N, D = x.shape
    E = edge_index.shape[1]
    HQ = D // 4
    NS, G = 16, 128

    src = edge_index[0].astype(jnp.int32)
    dst = edge_index[1].astype(jnp.int32)
    loop = jnp.arange(N, dtype=jnp.int32)

    E2 = E + N
    per_tile_groups = -(-E2 // (NS * G))
    NG = per_tile_groups + (per_tile_groups % 2)
    E2p = NS * NG * G
    pad = E2p - E2
    zi = jnp.zeros((pad,), jnp.int32)
    zf = jnp.zeros((pad,), x.dtype)
    src2 = jnp.concatenate([src, loop, zi]).reshape(NS, NG, G)
    dst2 = jnp.concatenate([dst, loop, zi]).reshape(NS, NG, G)
    ew2 = jnp.concatenate([edge_attr, jnp.ones((N,), x.dtype), zf]).reshape(NS, NG, G)

    NPAD = -(-N // 2048) * 2048

    xf = x.reshape(4 * N, HQ)
    h3 = _sc_propagate(xf, src2, dst2, ew2, N, HQ, NG, G, NPAD, K=3)

    BN = 1000
    out = pl.pallas_call(
        functools.partial(_mm_body, HQ=HQ),
        grid=(N // BN,),
        in_specs=[
            pl.BlockSpec((4, BN, HQ), lambda i: (0, i, 0)),
            pl.BlockSpec((D, D), lambda i: (0, 0)),
            pl.BlockSpec((1, D), lambda i: (0, 0)),
        ],
        out_specs=pl.BlockSpec((BN, D), lambda i: (i, 0)),
        out_shape=jax.ShapeDtypeStruct((N, D), jnp.float32),
    )(h3.reshape(4, NPAD, HQ), W, b.reshape(1, D))
    return out
